# baseline (speedup 1.0000x reference)
import jax
import jax.numpy as jnp
from jax.experimental import pallas as pl
from jax.experimental.pallas import tpu as pltpu

_FEM_WEIGHT = 1.0
_REG_WEIGHT = 0.1
_CHUNKS = 4


def _loss_kernel(nm_hbm, fm_hbm, pr_hbm, out_ref, nm_v, fm_v, pr_v, sems):
    n = nm_v.shape[0]
    rows = n // _CHUNKS

    copies = []
    for c in range(_CHUNKS):
        sl = pl.ds(c * rows, rows)
        cp_nm = pltpu.make_async_copy(nm_hbm.at[sl], nm_v.at[sl], sems.at[0, c])
        cp_fm = pltpu.make_async_copy(fm_hbm.at[sl], fm_v.at[sl], sems.at[1, c])
        cp_pr = pltpu.make_async_copy(pr_hbm.at[sl], pr_v.at[sl], sems.at[2, c])
        cp_nm.start()
        cp_fm.start()
        cp_pr.start()
        copies.append((cp_nm, cp_fm, cp_pr))

    fem = 0.0
    reg = 0.0
    for c in range(_CHUNKS):
        sl = pl.ds(c * rows, rows)
        cp_nm, cp_fm, cp_pr = copies[c]
        cp_nm.wait()
        cp_fm.wait()
        d = nm_v[sl] - fm_v[sl]
        fem = fem + jnp.sum(d * d)

        cp_pr.wait()
        p = pr_v[sl]
        core = p[:, :-1, :-1, :-1]
        dx = p[:, 1:, :-1, :-1] - core
        dy = p[:, :-1, 1:, :-1] - core
        dz = p[:, :-1, :-1, 1:] - core
        reg = reg + jnp.sum(dx * dx) + jnp.sum(dy * dy) + jnp.sum(dz * dz)

    n_total = 1.0
    for s in nm_v.shape:
        n_total *= s
    n_bc = n
    out_ref[0, 0] = fem * (_FEM_WEIGHT / n_total) + reg * (_REG_WEIGHT / n_bc)


def kernel(network_mesh, pc, fem_mesh, pred):
    del pc
    B, C, X, Y, Z = network_mesh.shape
    n = B * C
    nm = network_mesh.reshape(n, X, Y, Z)
    fm = fem_mesh.reshape(n, X, Y, Z)
    pr = pred.reshape(n, X, Y, Z)
    any_spec = pl.BlockSpec(memory_space=pl.ANY)
    out = pl.pallas_call(
        _loss_kernel,
        out_shape=jax.ShapeDtypeStruct((1, 1), jnp.float32),
        in_specs=[any_spec, any_spec, any_spec],
        out_specs=pl.BlockSpec(memory_space=pltpu.SMEM),
        scratch_shapes=[
            pltpu.VMEM((n, X, Y, Z), jnp.float32),
            pltpu.VMEM((n, X, Y, Z), jnp.float32),
            pltpu.VMEM((n, X, Y, Z), jnp.float32),
            pltpu.SemaphoreType.DMA((3, _CHUNKS)),
        ],
    )(nm, fm, pr)
    return out[0, 0]


# speedup vs baseline: 1.1121x; 1.1121x over previous
import jax
import jax.numpy as jnp
from jax.experimental import pallas as pl
from jax.experimental.pallas import tpu as pltpu

_FEM_WEIGHT = 1.0
_REG_WEIGHT = 0.1
_FM_CHUNKS = 4


def _loss_kernel(nm_hbm, fm_hbm, pr_hbm, out_ref, nm_v, fm_v, pr_v, sems):
    n = nm_v.shape[0]
    rows = n // _FM_CHUNKS

    cp_pr = pltpu.make_async_copy(pr_hbm, pr_v, sems.at[0])
    cp_nm = pltpu.make_async_copy(nm_hbm, nm_v, sems.at[1])
    cp_pr.start()
    cp_nm.start()
    cp_fm = []
    for c in range(_FM_CHUNKS):
        sl = pl.ds(c * rows, rows)
        cp = pltpu.make_async_copy(fm_hbm.at[sl], fm_v.at[sl], sems.at[2 + c])
        cp.start()
        cp_fm.append(cp)

    cp_pr.wait()
    p = pr_v[...]
    core = p[:, :-1, :-1, :-1]
    dx = p[:, 1:, :-1, :-1] - core
    dy = p[:, :-1, 1:, :-1] - core
    dz = p[:, :-1, :-1, 1:] - core
    reg = jnp.sum(dx * dx) + jnp.sum(dy * dy) + jnp.sum(dz * dz)

    cp_nm.wait()
    fem = 0.0
    for c in range(_FM_CHUNKS):
        sl = pl.ds(c * rows, rows)
        cp_fm[c].wait()
        d = nm_v[sl] - fm_v[sl]
        fem = fem + jnp.sum(d * d)

    n_total = 1.0
    for s in nm_v.shape:
        n_total *= s
    n_bc = n
    out_ref[0, 0] = fem * (_FEM_WEIGHT / n_total) + reg * (_REG_WEIGHT / n_bc)


def kernel(network_mesh, pc, fem_mesh, pred):
    del pc
    B, C, X, Y, Z = network_mesh.shape
    n = B * C
    nm = network_mesh.reshape(n, X, Y, Z)
    fm = fem_mesh.reshape(n, X, Y, Z)
    pr = pred.reshape(n, X, Y, Z)
    any_spec = pl.BlockSpec(memory_space=pl.ANY)
    out = pl.pallas_call(
        _loss_kernel,
        out_shape=jax.ShapeDtypeStruct((1, 1), jnp.float32),
        in_specs=[any_spec, any_spec, any_spec],
        out_specs=pl.BlockSpec(memory_space=pltpu.SMEM),
        scratch_shapes=[
            pltpu.VMEM((n, X, Y, Z), jnp.float32),
            pltpu.VMEM((n, X, Y, Z), jnp.float32),
            pltpu.VMEM((n, X, Y, Z), jnp.float32),
            pltpu.SemaphoreType.DMA((2 + _FM_CHUNKS,)),
        ],
    )(nm, fm, pr)
    return out[0, 0]
